# baseline (speedup 1.0000x reference)
import functools
import math

import jax
import jax.numpy as jnp
from jax import lax
from jax.experimental import pallas as pl
from jax.experimental.pallas import tpu as pltpu
from jax.experimental.pallas import tpu_sc as plsc

VOCAB = 100000
EMB = 300
DM = 512
B = 1024
L = 200
N_TOK = B * L
SCALE = math.sqrt(DM)


BM = 2000


def _proj_body(wv_ref, w_ref, out_ref):
    out_ref[...] = lax.dot_general(
        wv_ref[...], w_ref[...],
        dimension_numbers=(((1,), (1,)), ((), ())),
        preferred_element_type=jnp.float32,
    )


def _project_table(word_vectors, W):
    wv16 = word_vectors.astype(jnp.bfloat16)
    w16 = (W * SCALE).astype(jnp.bfloat16)
    return pl.pallas_call(
        _proj_body,
        grid=(VOCAB // BM,),
        in_specs=[
            pl.BlockSpec((BM, EMB), lambda i: (i, 0)),
            pl.BlockSpec((DM, EMB), lambda i: (0, 0)),
        ],
        out_specs=pl.BlockSpec((BM, DM), lambda i: (i, 0)),
        out_shape=jax.ShapeDtypeStruct((VOCAB, DM), jnp.float32),
    )(wv16, w16)


_INFO = plsc.get_sparse_core_info()
NC = _INFO.num_cores
NS = _INFO.num_subcores
NW = NC * NS
B_PER_W = N_TOK // NW
CHUNK = 80
NITER = B_PER_W // CHUNK
NBUF = 2


def _gather_sc(table, idx):
    mesh = plsc.VectorSubcoreMesh(core_axis_name="c", subcore_axis_name="s")

    @functools.partial(
        pl.kernel,
        mesh=mesh,
        out_type=jax.ShapeDtypeStruct((N_TOK, DM), jnp.float32),
        scratch_types=[
            pltpu.VMEM((B_PER_W,), jnp.int32),
            pltpu.VMEM((NBUF, CHUNK, DM), jnp.float32),
            pltpu.SemaphoreType.DMA,
            pltpu.SemaphoreType.DMA,
        ],
    )
    def k(table_hbm, idx_hbm, out_hbm, idx_v, rows_v, gsem0, gsem1):
        wid = lax.axis_index("s") * NC + lax.axis_index("c")
        base = wid * B_PER_W
        pltpu.sync_copy(idx_hbm.at[pl.ds(base, B_PER_W)], idx_v)
        gsems = (gsem0, gsem1)

        def start_gather(i, buf):
            pltpu.async_copy(
                table_hbm.at[idx_v.at[pl.ds(i * CHUNK, CHUNK)]],
                rows_v.at[buf],
                gsems[buf],
            )

        def wait_gather(buf):
            pltpu.make_async_copy(
                table_hbm.at[idx_v.at[pl.ds(0, CHUNK)]],
                rows_v.at[buf],
                gsems[buf],
            ).wait()

        for b in range(NBUF):
            start_gather(b, b)

        def body(j, _):
            for b in range(NBUF):
                i = j * NBUF + b
                wait_gather(b)
                pltpu.sync_copy(
                    rows_v.at[b],
                    out_hbm.at[pl.ds(base + i * CHUNK, CHUNK)],
                )

                @pl.when(i + NBUF < NITER)
                def _():
                    start_gather(i + NBUF, b)
            return 0

        lax.fori_loop(0, NITER // NBUF, body, 0)

    return k(table, idx)


def kernel(x, word_vectors, W):
    proj = _project_table(word_vectors, W)
    flat = _gather_sc(proj, x.reshape(-1))
    return flat.reshape(B, L, DM)


# speedup vs baseline: 1.3156x; 1.3156x over previous
import functools
import math

import jax
import jax.numpy as jnp
from jax import lax
from jax.experimental import pallas as pl
from jax.experimental.pallas import tpu as pltpu
from jax.experimental.pallas import tpu_sc as plsc

VOCAB = 100000
EMB = 300
DM = 512
B = 1024
L = 200
N_TOK = B * L
SCALE = math.sqrt(DM)


BM = 2048


def _proj_body(wvt_ref, w_ref, out_ref):
    out_ref[...] = lax.dot_general(
        wvt_ref[...], w_ref[...],
        dimension_numbers=(((0,), (1,)), ((), ())),
        preferred_element_type=jnp.float32,
    ) * SCALE


def _project_table(word_vectors, W):
    wvt = word_vectors.T
    return pl.pallas_call(
        _proj_body,
        grid=((VOCAB + BM - 1) // BM,),
        in_specs=[
            pl.BlockSpec((EMB, BM), lambda i: (0, i)),
            pl.BlockSpec((DM, EMB), lambda i: (0, 0)),
        ],
        out_specs=pl.BlockSpec((BM, DM), lambda i: (i, 0)),
        out_shape=jax.ShapeDtypeStruct((VOCAB, DM), jnp.float32),
    )(wvt, W)


_INFO = plsc.get_sparse_core_info()
NC = _INFO.num_cores
NS = _INFO.num_subcores
NW = NC * NS
B_PER_W = N_TOK // NW
CHUNK = 80
NITER = B_PER_W // CHUNK
NBUF = 2


def _gather_sc(table, idx):
    mesh = plsc.VectorSubcoreMesh(core_axis_name="c", subcore_axis_name="s")

    @functools.partial(
        pl.kernel,
        mesh=mesh,
        out_type=jax.ShapeDtypeStruct((N_TOK, DM), jnp.float32),
        scratch_types=[
            pltpu.VMEM((B_PER_W,), jnp.int32),
            pltpu.VMEM((NBUF, CHUNK, DM), jnp.float32),
            pltpu.SemaphoreType.DMA,
            pltpu.SemaphoreType.DMA,
        ],
    )
    def k(table_hbm, idx_hbm, out_hbm, idx_v, rows_v, gsem0, gsem1):
        wid = lax.axis_index("s") * NC + lax.axis_index("c")
        base = wid * B_PER_W
        pltpu.sync_copy(idx_hbm.at[pl.ds(base, B_PER_W)], idx_v)
        gsems = (gsem0, gsem1)

        def start_gather(i, buf):
            pltpu.async_copy(
                table_hbm.at[idx_v.at[pl.ds(i * CHUNK, CHUNK)]],
                rows_v.at[buf],
                gsems[buf],
            )

        def wait_gather(buf):
            pltpu.make_async_copy(
                table_hbm.at[idx_v.at[pl.ds(0, CHUNK)]],
                rows_v.at[buf],
                gsems[buf],
            ).wait()

        for b in range(NBUF):
            start_gather(b, b)

        def body(j, _):
            for b in range(NBUF):
                i = j * NBUF + b
                wait_gather(b)
                pltpu.sync_copy(
                    rows_v.at[b],
                    out_hbm.at[pl.ds(base + i * CHUNK, CHUNK)],
                )

                @pl.when(i + NBUF < NITER)
                def _():
                    start_gather(i + NBUF, b)
            return 0

        lax.fori_loop(0, NITER // NBUF, body, 0)

    return k(table, idx)


def kernel(x, word_vectors, W):
    proj = _project_table(word_vectors, W)
    flat = _gather_sc(proj, x.reshape(-1))
    return flat.reshape(B, L, DM)
